# baseline (speedup 1.0000x reference)
import functools

import jax
import jax.numpy as jnp
from jax import lax
from jax.experimental import pallas as pl
from jax.experimental.pallas import tpu as pltpu
from jax.experimental.pallas import tpu_sc as plsc

B, S, D, MAX_LEN = 4, 4096, 768, 8192
LANES = 16
NUM_WORKERS = 32
ROWS_PER_WORKER = (B * S) // NUM_WORKERS
CHUNK = 32
NCHUNKS = ROWS_PER_WORKER // CHUNK
NBUF = 2
VECS_PER_ROW = D // LANES
PE_ROWS = CHUNK + 8
ZROW = PE_ROWS


def _sc_kernel(x_hbm, len_hbm, pe_hbm, out_hbm,
               len_v, xb, peb, sem_in, sem_pe, sem_out):
    cid = lax.axis_index("c")
    sid = lax.axis_index("s")
    wid = sid * 2 + cid

    pltpu.sync_copy(len_hbm, len_v)
    lane = lax.iota(jnp.int32, 16)
    len_vec = len_v[...]
    dnums = lax.GatherDimensionNumbers(
        offset_dims=(), collapsed_slice_dims=(0,), start_index_map=(0,))
    lengths4 = []
    for bb in range(B):
        idx = jnp.where(lane == 0, bb, lane)
        picked = lax.gather(
            len_vec, idx[:, None], dnums, (1,),
            mode=lax.GatherScatterMode.PROMISE_IN_BOUNDS)
        lengths4.append(jnp.squeeze(lax.slice(picked, (0,), (1,))))

    def chunk_geom(c):
        g = wid + NUM_WORKERS * c
        row0 = g * CHUNK
        s0 = row0 % S
        bsel = row0 // S
        length = lengths4[B - 1]
        for bb in range(B - 2, -1, -1):
            length = jnp.where(bsel == bb, lengths4[bb], length)
        return row0, length - s0

    zeros = jnp.zeros((LANES,), jnp.float32)
    for buf in range(NBUF):
        for j in range(VECS_PER_ROW):
            peb[buf, ZROW, pl.ds(j * LANES, LANES)] = zeros

    d_in = [None] * NCHUNKS
    d_out = [None] * NCHUNKS

    def pe_base(c):
        _, hi = chunk_geom(c)
        base = jnp.maximum(hi - (CHUNK - 1), 0)
        return hi, pl.multiple_of((base // 8) * 8, 8)

    def pe_stream(c, buf):
        _, base_al = pe_base(c)
        return pltpu.make_async_copy(
            pe_hbm.at[pl.ds(base_al, PE_ROWS)],
            peb.at[buf, pl.ds(0, PE_ROWS)],
            sem_pe.at[buf])

    def stage_in(c):
        buf = c % NBUF
        row0, hi = chunk_geom(c)
        if c >= NBUF:
            d_out[c - NBUF].wait()
        d_in[c] = pltpu.async_copy(
            x_hbm.at[pl.ds(row0, CHUNK)], xb.at[buf], sem_in.at[buf])

        @pl.when(hi >= 1)
        def _():
            pe_stream(c, buf).start()

    def stage_add(c):
        buf = c % NBUF
        row0, _ = chunk_geom(c)
        hi, base_al = pe_base(c)

        d_in[c].wait()

        @pl.when(hi >= CHUNK)
        def _():
            pe_stream(c, buf).wait()
            off = hi - base_al

            def row_body(r):
                src = off - r

                def vec_body(o):
                    sl = pl.ds(o, LANES)
                    plsc.addupdate(xb.at[buf, r, sl], peb[buf, src, sl])

                plsc.parallel_loop(0, D, LANES, unroll=16)(vec_body)

            plsc.parallel_loop(0, CHUNK, unroll=2)(row_body)

        @pl.when((hi >= 1) & (hi < CHUNK))
        def _():
            pe_stream(c, buf).wait()

            def row_body(r):
                src = jnp.where(hi - r >= 1, hi - r - base_al, ZROW)

                def vec_body(o):
                    sl = pl.ds(o, LANES)
                    plsc.addupdate(xb.at[buf, r, sl], peb[buf, src, sl])

                plsc.parallel_loop(0, D, LANES, unroll=16)(vec_body)

            plsc.parallel_loop(0, CHUNK, unroll=2)(row_body)

        d_out[c] = pltpu.async_copy(
            xb.at[buf], out_hbm.at[pl.ds(row0, CHUNK)], sem_out.at[buf])

    for c in range(NCHUNKS + 1):
        if c < NCHUNKS:
            stage_in(c)
        if c >= 1:
            stage_add(c - 1)
    for c in range(NCHUNKS - NBUF, NCHUNKS):
        d_out[c].wait()


def kernel(x, lengths, pe):
    n_batch, n_seq, d_emb = x.shape
    xf = x.reshape(n_batch * n_seq, d_emb)
    len_pad = jnp.zeros((16,), jnp.int32).at[:n_batch].set(lengths)

    mesh = plsc.VectorSubcoreMesh(core_axis_name="c", subcore_axis_name="s")
    run = functools.partial(
        pl.kernel,
        mesh=mesh,
        out_type=jax.ShapeDtypeStruct((n_batch * n_seq, d_emb), jnp.float32),
        scratch_types=[
            pltpu.VMEM((16,), jnp.int32),
            pltpu.VMEM((NBUF, CHUNK, D), jnp.float32),
            pltpu.VMEM((NBUF, PE_ROWS + 1, D), jnp.float32),
            pltpu.SemaphoreType.DMA((NBUF,)),
            pltpu.SemaphoreType.DMA((NBUF,)),
            pltpu.SemaphoreType.DMA((NBUF,)),
        ],
    )(_sc_kernel)
    out = run(xf, len_pad, pe)
    return out.reshape(n_batch, n_seq, d_emb)


# speedup vs baseline: 1.0564x; 1.0564x over previous
import functools

import jax
import jax.numpy as jnp
from jax import lax
from jax.experimental import pallas as pl
from jax.experimental.pallas import tpu as pltpu
from jax.experimental.pallas import tpu_sc as plsc

B, S, D, MAX_LEN = 4, 4096, 768, 8192
LANES = 16
NUM_WORKERS = 32
ROWS_PER_WORKER = (B * S) // NUM_WORKERS
CHUNK = 32
NCHUNKS = ROWS_PER_WORKER // CHUNK
NBUF = 2
VECS_PER_ROW = D // LANES
PE_ROWS = CHUNK + 8
ZROW = PE_ROWS


def _sc_kernel(x_hbm, len_hbm, pe_hbm, out_hbm,
               len_v, xb, peb, sem_in, sem_pe, sem_out):
    cid = lax.axis_index("c")
    sid = lax.axis_index("s")
    wid = sid * 2 + cid

    pltpu.sync_copy(len_hbm, len_v)
    lane = lax.iota(jnp.int32, 16)
    len_vec = len_v[...]
    dnums = lax.GatherDimensionNumbers(
        offset_dims=(), collapsed_slice_dims=(0,), start_index_map=(0,))
    lengths4 = []
    for bb in range(B):
        idx = jnp.where(lane == 0, bb, lane)
        picked = lax.gather(
            len_vec, idx[:, None], dnums, (1,),
            mode=lax.GatherScatterMode.PROMISE_IN_BOUNDS)
        lengths4.append(jnp.squeeze(lax.slice(picked, (0,), (1,))))

    def chunk_geom(c):
        g = wid + NUM_WORKERS * c
        row0 = g * CHUNK
        s0 = row0 % S
        bsel = row0 // S
        length = lengths4[B - 1]
        for bb in range(B - 2, -1, -1):
            length = jnp.where(bsel == bb, lengths4[bb], length)
        return row0, length - s0

    zeros = jnp.zeros((LANES,), jnp.float32)
    for buf in range(NBUF):
        for j in range(VECS_PER_ROW):
            peb[buf, ZROW, pl.ds(j * LANES, LANES)] = zeros

    d_in = [None] * NCHUNKS
    d_out = [None] * NCHUNKS

    def pe_base(c):
        _, hi = chunk_geom(c)
        base = jnp.maximum(hi - (CHUNK - 1), 0)
        return hi, pl.multiple_of((base // 8) * 8, 8)

    def pe_stream(c, buf):
        _, base_al = pe_base(c)
        return pltpu.make_async_copy(
            pe_hbm.at[pl.ds(base_al, PE_ROWS)],
            peb.at[buf, pl.ds(0, PE_ROWS)],
            sem_pe.at[buf])

    def stage_in(c):
        buf = c % NBUF
        row0, hi = chunk_geom(c)
        if c >= NBUF:
            d_out[c - NBUF].wait()
        d_in[c] = pltpu.async_copy(
            x_hbm.at[pl.ds(row0, CHUNK)], xb.at[buf], sem_in.at[buf])

        @pl.when(hi >= 1)
        def _():
            pe_stream(c, buf).start()

    def stage_add(c):
        buf = c % NBUF
        row0, _ = chunk_geom(c)
        hi, base_al = pe_base(c)

        d_in[c].wait()

        @pl.when(hi >= CHUNK)
        def _():
            pe_stream(c, buf).wait()
            off = hi - base_al

            def row_body(r):
                src = off - r

                def vec_body(o):
                    sl = pl.ds(o, LANES)
                    plsc.addupdate(xb.at[buf, r, sl], peb[buf, src, sl])

                plsc.parallel_loop(0, D, LANES, unroll=8)(vec_body)

            plsc.parallel_loop(0, CHUNK)(row_body)

        @pl.when((hi >= 1) & (hi < CHUNK))
        def _():
            pe_stream(c, buf).wait()

            def row_body(r):
                src = jnp.where(hi - r >= 1, hi - r - base_al, ZROW)

                def vec_body(o):
                    sl = pl.ds(o, LANES)
                    plsc.addupdate(xb.at[buf, r, sl], peb[buf, src, sl])

                plsc.parallel_loop(0, D, LANES, unroll=8)(vec_body)

            plsc.parallel_loop(0, CHUNK)(row_body)

        d_out[c] = pltpu.async_copy(
            xb.at[buf], out_hbm.at[pl.ds(row0, CHUNK)], sem_out.at[buf])

    for c in range(NCHUNKS + 1):
        if c < NCHUNKS:
            stage_in(c)
        if c >= 1:
            stage_add(c - 1)
    for c in range(NCHUNKS - NBUF, NCHUNKS):
        d_out[c].wait()


def kernel(x, lengths, pe):
    n_batch, n_seq, d_emb = x.shape
    xf = x.reshape(n_batch * n_seq, d_emb)
    len_pad = jnp.zeros((16,), jnp.int32).at[:n_batch].set(lengths)

    mesh = plsc.VectorSubcoreMesh(core_axis_name="c", subcore_axis_name="s")
    run = functools.partial(
        pl.kernel,
        mesh=mesh,
        out_type=jax.ShapeDtypeStruct((n_batch * n_seq, d_emb), jnp.float32),
        scratch_types=[
            pltpu.VMEM((16,), jnp.int32),
            pltpu.VMEM((NBUF, CHUNK, D), jnp.float32),
            pltpu.VMEM((NBUF, PE_ROWS + 1, D), jnp.float32),
            pltpu.SemaphoreType.DMA((NBUF,)),
            pltpu.SemaphoreType.DMA((NBUF,)),
            pltpu.SemaphoreType.DMA((NBUF,)),
        ],
    )(_sc_kernel)
    out = run(xf, len_pad, pe)
    return out.reshape(n_batch, n_seq, d_emb)


# speedup vs baseline: 1.0571x; 1.0006x over previous
import functools

import jax
import jax.numpy as jnp
from jax import lax
from jax.experimental import pallas as pl
from jax.experimental.pallas import tpu as pltpu
from jax.experimental.pallas import tpu_sc as plsc

B, S, D, MAX_LEN = 4, 4096, 768, 8192
LANES = 16
NUM_WORKERS = 32
ROWS_PER_WORKER = (B * S) // NUM_WORKERS
CHUNK = 32
NCHUNKS = ROWS_PER_WORKER // CHUNK
NBUF = 2
VECS_PER_ROW = D // LANES
PE_ROWS = CHUNK + 8
ZROW = PE_ROWS


def _sc_kernel(x_hbm, len_hbm, pe_hbm, out_hbm,
               len_v, xb, peb, sem_in, sem_pe, sem_out):
    cid = lax.axis_index("c")
    sid = lax.axis_index("s")
    wid = sid * 2 + cid

    pltpu.sync_copy(len_hbm, len_v)
    lane = lax.iota(jnp.int32, 16)
    len_vec = len_v[...]
    dnums = lax.GatherDimensionNumbers(
        offset_dims=(), collapsed_slice_dims=(0,), start_index_map=(0,))
    lengths4 = []
    for bb in range(B):
        idx = jnp.where(lane == 0, bb, lane)
        picked = lax.gather(
            len_vec, idx[:, None], dnums, (1,),
            mode=lax.GatherScatterMode.PROMISE_IN_BOUNDS)
        lengths4.append(jnp.squeeze(lax.slice(picked, (0,), (1,))))

    def chunk_geom(c):
        g = wid + NUM_WORKERS * c
        row0 = g * CHUNK
        s0 = row0 % S
        bsel = row0 // S
        length = lengths4[B - 1]
        for bb in range(B - 2, -1, -1):
            length = jnp.where(bsel == bb, lengths4[bb], length)
        return row0, length - s0

    zeros = jnp.zeros((LANES,), jnp.float32)
    for buf in range(NBUF):
        for j in range(VECS_PER_ROW):
            peb[buf, ZROW, pl.ds(j * LANES, LANES)] = zeros

    d_in = [None] * NCHUNKS
    d_out = [None] * NCHUNKS

    def pe_base(c):
        _, hi = chunk_geom(c)
        base = jnp.maximum(hi - (CHUNK - 1), 0)
        return hi, pl.multiple_of((base // 8) * 8, 8)

    def pe_stream(c, buf):
        _, base_al = pe_base(c)
        return pltpu.make_async_copy(
            pe_hbm.at[pl.ds(base_al, PE_ROWS)],
            peb.at[buf, pl.ds(0, PE_ROWS)],
            sem_pe.at[buf])

    def stage_in(c):
        buf = c % NBUF
        row0, hi = chunk_geom(c)
        if c >= NBUF:
            d_out[c - NBUF].wait()
        @pl.when(hi >= 1)
        def _():
            pe_stream(c, buf).start()

        d_in[c] = pltpu.async_copy(
            x_hbm.at[pl.ds(row0, CHUNK)], xb.at[buf], sem_in.at[buf])

    def stage_add(c):
        buf = c % NBUF
        row0, _ = chunk_geom(c)
        hi, base_al = pe_base(c)

        d_in[c].wait()

        @pl.when(hi >= CHUNK)
        def _():
            pe_stream(c, buf).wait()
            off = hi - base_al

            def row_body(r):
                src = off - r

                def vec_body(o):
                    sl = pl.ds(o, LANES)
                    plsc.addupdate(xb.at[buf, r, sl], peb[buf, src, sl])

                plsc.parallel_loop(0, D, LANES, unroll=8)(vec_body)

            plsc.parallel_loop(0, CHUNK)(row_body)

        @pl.when((hi >= 1) & (hi < CHUNK))
        def _():
            pe_stream(c, buf).wait()

            def row_body(r):
                src = jnp.where(hi - r >= 1, hi - r - base_al, ZROW)

                def vec_body(o):
                    sl = pl.ds(o, LANES)
                    plsc.addupdate(xb.at[buf, r, sl], peb[buf, src, sl])

                plsc.parallel_loop(0, D, LANES, unroll=8)(vec_body)

            plsc.parallel_loop(0, CHUNK)(row_body)

        d_out[c] = pltpu.async_copy(
            xb.at[buf], out_hbm.at[pl.ds(row0, CHUNK)], sem_out.at[buf])

    for c in range(NCHUNKS + 1):
        if c < NCHUNKS:
            stage_in(c)
        if c >= 1:
            stage_add(c - 1)
    for c in range(NCHUNKS - NBUF, NCHUNKS):
        d_out[c].wait()


def kernel(x, lengths, pe):
    n_batch, n_seq, d_emb = x.shape
    xf = x.reshape(n_batch * n_seq, d_emb)
    len_pad = jnp.zeros((16,), jnp.int32).at[:n_batch].set(lengths)

    mesh = plsc.VectorSubcoreMesh(core_axis_name="c", subcore_axis_name="s")
    run = functools.partial(
        pl.kernel,
        mesh=mesh,
        out_type=jax.ShapeDtypeStruct((n_batch * n_seq, d_emb), jnp.float32),
        scratch_types=[
            pltpu.VMEM((16,), jnp.int32),
            pltpu.VMEM((NBUF, CHUNK, D), jnp.float32),
            pltpu.VMEM((NBUF, PE_ROWS + 1, D), jnp.float32),
            pltpu.SemaphoreType.DMA((NBUF,)),
            pltpu.SemaphoreType.DMA((NBUF,)),
            pltpu.SemaphoreType.DMA((NBUF,)),
        ],
    )(_sc_kernel)
    out = run(xf, len_pad, pe)
    return out.reshape(n_batch, n_seq, d_emb)
